# baseline (speedup 1.0000x reference)
import functools

import jax
import jax.numpy as jnp
from jax import lax
from jax.experimental import pallas as pl
from jax.experimental.pallas import tpu as pltpu
from jax.experimental.pallas import tpu_sc as plsc

DIM = 33
D2 = DIM * DIM
D3 = DIM * DIM * DIM
LUT_PAD = ((3 * D3 + 7) // 8) * 8
NC, NS, L = 2, 16, 16
NW = NC * NS
CHUNK = 1024


def _make_sc_kernel(n_rows, n_px):
    assert NW % (n_rows // 3) == 0
    w_per_img = NW // (n_rows // 3)
    px_per_w = n_px // w_per_img
    assert px_per_w % (2 * CHUNK) == 0
    n_chunks = px_per_w // CHUNK

    mesh = plsc.VectorSubcoreMesh(
        core_axis_name="c", subcore_axis_name="s",
        num_cores=NC, num_subcores=NS)

    @functools.partial(
        pl.kernel,
        out_type=jax.ShapeDtypeStruct((n_rows * n_px,), jnp.float32),
        mesh=mesh,
        scratch_types=[
            pltpu.VMEM((LUT_PAD,), jnp.float32),
            pltpu.VMEM((3 * CHUNK,), jnp.float32),
            pltpu.VMEM((3 * CHUNK,), jnp.float32),
            pltpu.VMEM((3 * CHUNK,), jnp.float32),
            pltpu.VMEM((3 * CHUNK,), jnp.float32),
            pltpu.SemaphoreType.DMA,
            pltpu.SemaphoreType.DMA,
            pltpu.SemaphoreType.DMA,
            pltpu.SemaphoreType.DMA,
        ],
        compiler_params=pltpu.CompilerParams(needs_layout_passes=False),
    )
    def sc_kernel(x_hbm, lut_hbm, out_hbm, lut_v,
                  in0, in1, ob0, ob1, is0, is1, os0, os1):
        wid = lax.axis_index("s") * NC + lax.axis_index("c")
        img = wid // w_per_img
        plane = img * 3 * n_px
        base_px = (wid % w_per_img) * px_per_w

        pltpu.sync_copy(lut_hbm, lut_v)

        ins = (in0, in1)
        obs = (ob0, ob1)
        in_sems = (is0, is1)
        out_sems = (os0, os1)
        scale = jnp.float32(DIM - 1)

        def issue_in(ci, buf, sem):
            off = plane + base_px + ci * CHUNK
            for c in range(3):
                pltpu.async_copy(
                    x_hbm.at[pl.ds(off + c * n_px, CHUNK)],
                    buf.at[pl.ds(c * CHUNK, CHUNK)], sem)

        def wait_in(buf, sem):
            for c in range(3):
                pltpu.make_async_copy(
                    x_hbm.at[pl.ds(0, CHUNK)],
                    buf.at[pl.ds(c * CHUNK, CHUNK)], sem).wait()

        def issue_out(ci, buf, sem):
            off = plane + base_px + ci * CHUNK
            for c in range(3):
                pltpu.async_copy(
                    buf.at[pl.ds(c * CHUNK, CHUNK)],
                    out_hbm.at[pl.ds(off + c * n_px, CHUNK)], sem)

        def wait_out(buf, sem):
            for c in range(3):
                pltpu.make_async_copy(
                    buf.at[pl.ds(c * CHUNK, CHUNK)],
                    out_hbm.at[pl.ds(0, CHUNK)], sem).wait()

        def compute(xin_v, out_v):
            def do_vreg(s):
                def prep(v):
                    p = jnp.minimum(jnp.maximum(v, 0.0), 1.0) * scale
                    i = jnp.minimum(p.astype(jnp.int32), DIM - 2)
                    return i, p - i.astype(jnp.float32)

                ir, fr = prep(xin_v[pl.ds(s, L)])
                ig, fg = prep(xin_v[pl.ds(CHUNK + s, L)])
                ib, fb = prep(xin_v[pl.ds(2 * CHUNK + s, L)])
                base = ib * D2 + ig * DIM + ir
                acc0 = jnp.zeros((L,), jnp.float32)
                acc1 = jnp.zeros((L,), jnp.float32)
                acc2 = jnp.zeros((L,), jnp.float32)
                for db, wb in ((0, 1.0 - fb), (1, fb)):
                    for dg, wg in ((0, 1.0 - fg), (1, fg)):
                        wbg = wb * wg
                        for dr, wr in ((0, 1.0 - fr), (1, fr)):
                            w = wbg * wr
                            idx = base + (db * D2 + dg * DIM + dr)
                            acc0 += w * plsc.load_gather(lut_v, [idx])
                            acc1 += w * plsc.load_gather(lut_v, [idx + D3])
                            acc2 += w * plsc.load_gather(
                                lut_v, [idx + 2 * D3])
                out_v[pl.ds(s, L)] = acc0
                out_v[pl.ds(CHUNK + s, L)] = acc1
                out_v[pl.ds(2 * CHUNK + s, L)] = acc2

            def vec_body(vi, _):
                s0 = vi * (2 * L)
                do_vreg(s0)
                do_vreg(s0 + L)
                return 0

            lax.fori_loop(0, CHUNK // (2 * L), vec_body, 0)

        issue_in(0, in0, is0)

        def body2(ci2, _):
            for b in range(2):
                ci = ci2 * 2 + b

                @pl.when(ci + 1 < n_chunks)
                def _():
                    issue_in(ci + 1, ins[1 - b], in_sems[1 - b])

                wait_in(ins[b], in_sems[b])

                @pl.when(ci2 >= 1)
                def _():
                    wait_out(obs[b], out_sems[b])

                compute(ins[b], obs[b])
                issue_out(ci, obs[b], out_sems[b])
            return 0

        lax.fori_loop(0, n_chunks // 2, body2, 0)
        wait_out(ob0, os0)
        wait_out(ob1, os1)

    return sc_kernel


def kernel(x, LUT):
    B, C, H, W = x.shape
    n_px = H * W
    xr = x.reshape(B * C * n_px)
    lut_flat = LUT.reshape(C * LUT.shape[1] ** 3)
    lut_pad = jnp.pad(lut_flat, (0, LUT_PAD - lut_flat.shape[0]))
    out = _make_sc_kernel(B * C, n_px)(xr, lut_pad)
    return out.reshape(B, C, H, W)


# speedup vs baseline: 1.0395x; 1.0395x over previous
import functools

import jax
import jax.numpy as jnp
from jax import lax
from jax.experimental import pallas as pl
from jax.experimental.pallas import tpu as pltpu
from jax.experimental.pallas import tpu_sc as plsc

DIM = 33
D2 = DIM * DIM
D3 = DIM * DIM * DIM
LUT_PAD = ((3 * D3 + 7) // 8) * 8
NC, NS, L = 2, 16, 16
NW = NC * NS
CHUNK = 2048


def _make_sc_kernel(n_rows, n_px):
    assert NW % (n_rows // 3) == 0
    w_per_img = NW // (n_rows // 3)
    px_per_w = n_px // w_per_img
    assert px_per_w % (2 * CHUNK) == 0
    n_chunks = px_per_w // CHUNK

    mesh = plsc.VectorSubcoreMesh(
        core_axis_name="c", subcore_axis_name="s",
        num_cores=NC, num_subcores=NS)

    @functools.partial(
        pl.kernel,
        out_type=jax.ShapeDtypeStruct((n_rows * n_px,), jnp.float32),
        mesh=mesh,
        scratch_types=[
            pltpu.VMEM((LUT_PAD,), jnp.float32),
            pltpu.VMEM((3 * CHUNK,), jnp.float32),
            pltpu.VMEM((3 * CHUNK,), jnp.float32),
            pltpu.VMEM((3 * CHUNK,), jnp.float32),
            pltpu.SemaphoreType.DMA,
            pltpu.SemaphoreType.DMA,
            pltpu.SemaphoreType.DMA,
        ],
        compiler_params=pltpu.CompilerParams(needs_layout_passes=False),
    )
    def sc_kernel(x_hbm, lut_hbm, out_hbm, lut_v,
                  in0, in1, ob0, is0, is1, os0):
        wid = lax.axis_index("s") * NC + lax.axis_index("c")
        img = wid // w_per_img
        plane = img * 3 * n_px
        base_px = (wid % w_per_img) * px_per_w

        pltpu.sync_copy(lut_hbm, lut_v)

        ins = (in0, in1)
        in_sems = (is0, is1)
        scale = jnp.float32(DIM - 1)

        def issue_in(ci, buf, sem):
            off = plane + base_px + ci * CHUNK
            for c in range(3):
                pltpu.async_copy(
                    x_hbm.at[pl.ds(off + c * n_px, CHUNK)],
                    buf.at[pl.ds(c * CHUNK, CHUNK)], sem)

        def wait_in(buf, sem):
            for c in range(3):
                pltpu.make_async_copy(
                    x_hbm.at[pl.ds(0, CHUNK)],
                    buf.at[pl.ds(c * CHUNK, CHUNK)], sem).wait()

        def issue_out(ci, buf, sem):
            off = plane + base_px + ci * CHUNK
            for c in range(3):
                pltpu.async_copy(
                    buf.at[pl.ds(c * CHUNK, CHUNK)],
                    out_hbm.at[pl.ds(off + c * n_px, CHUNK)], sem)

        def wait_out(buf, sem):
            for c in range(3):
                pltpu.make_async_copy(
                    buf.at[pl.ds(c * CHUNK, CHUNK)],
                    out_hbm.at[pl.ds(0, CHUNK)], sem).wait()

        def compute(xin_v, out_v):
            def do_vreg(s):
                def prep(v):
                    p = jnp.minimum(jnp.maximum(v, 0.0), 1.0) * scale
                    i = jnp.minimum(p.astype(jnp.int32), DIM - 2)
                    return i, p - i.astype(jnp.float32)

                ir, fr = prep(xin_v[pl.ds(s, L)])
                ig, fg = prep(xin_v[pl.ds(CHUNK + s, L)])
                ib, fb = prep(xin_v[pl.ds(2 * CHUNK + s, L)])
                base = ib * D2 + ig * DIM + ir
                acc0 = jnp.zeros((L,), jnp.float32)
                acc1 = jnp.zeros((L,), jnp.float32)
                acc2 = jnp.zeros((L,), jnp.float32)
                for db, wb in ((0, 1.0 - fb), (1, fb)):
                    for dg, wg in ((0, 1.0 - fg), (1, fg)):
                        wbg = wb * wg
                        for dr, wr in ((0, 1.0 - fr), (1, fr)):
                            w = wbg * wr
                            idx = base + (db * D2 + dg * DIM + dr)
                            acc0 += w * plsc.load_gather(lut_v, [idx])
                            acc1 += w * plsc.load_gather(lut_v, [idx + D3])
                            acc2 += w * plsc.load_gather(
                                lut_v, [idx + 2 * D3])
                out_v[pl.ds(s, L)] = acc0
                out_v[pl.ds(CHUNK + s, L)] = acc1
                out_v[pl.ds(2 * CHUNK + s, L)] = acc2

            def vec_body(vi, _):
                do_vreg(vi * L)
                return 0

            lax.fori_loop(0, CHUNK // L, vec_body, 0)

        issue_in(0, in0, is0)

        def body2(ci2, _):
            for b in range(2):
                ci = ci2 * 2 + b

                @pl.when(ci + 1 < n_chunks)
                def _():
                    issue_in(ci + 1, ins[1 - b], in_sems[1 - b])

                wait_in(ins[b], in_sems[b])

                @pl.when(ci >= 1)
                def _():
                    wait_out(ob0, os0)

                compute(ins[b], ob0)
                issue_out(ci, ob0, os0)
            return 0

        lax.fori_loop(0, n_chunks // 2, body2, 0)
        wait_out(ob0, os0)

    return sc_kernel


def kernel(x, LUT):
    B, C, H, W = x.shape
    n_px = H * W
    xr = x.reshape(B * C * n_px)
    lut_flat = LUT.reshape(C * LUT.shape[1] ** 3)
    lut_pad = jnp.pad(lut_flat, (0, LUT_PAD - lut_flat.shape[0]))
    out = _make_sc_kernel(B * C, n_px)(xr, lut_pad)
    return out.reshape(B, C, H, W)


# speedup vs baseline: 1.0782x; 1.0373x over previous
import functools

import jax
import jax.numpy as jnp
from jax import lax
from jax.experimental import pallas as pl
from jax.experimental.pallas import tpu as pltpu
from jax.experimental.pallas import tpu_sc as plsc

DIM = 33
D2 = DIM * DIM
D3 = DIM * DIM * DIM
LUT_PAD = ((3 * D3 + 7) // 8) * 8
NC, NS, L = 2, 16, 16
NW = NC * NS
CHUNK = 1024


def _make_sc_kernel(n_rows, n_px):
    assert NW % (n_rows // 3) == 0
    w_per_img = NW // (n_rows // 3)
    px_per_w = n_px // w_per_img
    assert px_per_w % (2 * CHUNK) == 0
    n_chunks = px_per_w // CHUNK

    mesh = plsc.VectorSubcoreMesh(
        core_axis_name="c", subcore_axis_name="s",
        num_cores=NC, num_subcores=NS)

    @functools.partial(
        pl.kernel,
        out_type=jax.ShapeDtypeStruct((n_rows * n_px,), jnp.float32),
        mesh=mesh,
        scratch_types=[
            pltpu.VMEM((LUT_PAD,), jnp.float32),
            pltpu.VMEM((3 * CHUNK,), jnp.float32),
            pltpu.VMEM((3 * CHUNK,), jnp.float32),
            pltpu.VMEM((3 * CHUNK,), jnp.float32),
            pltpu.VMEM((3 * CHUNK,), jnp.float32),
            pltpu.SemaphoreType.DMA,
            pltpu.SemaphoreType.DMA,
            pltpu.SemaphoreType.DMA,
            pltpu.SemaphoreType.DMA,
        ],
        compiler_params=pltpu.CompilerParams(needs_layout_passes=False),
    )
    def sc_kernel(x_hbm, lut_hbm, out_hbm, lut_v,
                  in0, in1, ob0, ob1, is0, is1, os0, os1):
        wid = lax.axis_index("s") * NC + lax.axis_index("c")
        img = wid // w_per_img
        plane = img * 3 * n_px
        base_px = (wid % w_per_img) * px_per_w

        pltpu.sync_copy(lut_hbm, lut_v)

        ins = (in0, in1)
        obs = (ob0, ob1)
        in_sems = (is0, is1)
        out_sems = (os0, os1)
        scale = jnp.float32(DIM - 1)

        def issue_in(ci, buf, sem):
            off = plane + base_px + ci * CHUNK
            for c in range(3):
                pltpu.async_copy(
                    x_hbm.at[pl.ds(off + c * n_px, CHUNK)],
                    buf.at[pl.ds(c * CHUNK, CHUNK)], sem)

        def wait_in(buf, sem):
            for c in range(3):
                pltpu.make_async_copy(
                    x_hbm.at[pl.ds(0, CHUNK)],
                    buf.at[pl.ds(c * CHUNK, CHUNK)], sem).wait()

        def issue_out(ci, buf, sem):
            off = plane + base_px + ci * CHUNK
            for c in range(3):
                pltpu.async_copy(
                    buf.at[pl.ds(c * CHUNK, CHUNK)],
                    out_hbm.at[pl.ds(off + c * n_px, CHUNK)], sem)

        def wait_out(buf, sem):
            for c in range(3):
                pltpu.make_async_copy(
                    buf.at[pl.ds(c * CHUNK, CHUNK)],
                    out_hbm.at[pl.ds(0, CHUNK)], sem).wait()

        def compute(xin_v, out_v):
            def do_vreg(s):
                def prep(v):
                    p = jnp.minimum(jnp.maximum(v, 0.0), 1.0) * scale
                    i = jnp.minimum(p.astype(jnp.int32), DIM - 2)
                    return i, p - i.astype(jnp.float32)

                ir, fr = prep(xin_v[pl.ds(s, L)])
                ig, fg = prep(xin_v[pl.ds(CHUNK + s, L)])
                ib, fb = prep(xin_v[pl.ds(2 * CHUNK + s, L)])
                base = ib * D2 + ig * DIM + ir
                acc0 = jnp.zeros((L,), jnp.float32)
                acc1 = jnp.zeros((L,), jnp.float32)
                acc2 = jnp.zeros((L,), jnp.float32)
                for db, wb in ((0, 1.0 - fb), (1, fb)):
                    for dg, wg in ((0, 1.0 - fg), (1, fg)):
                        wbg = wb * wg
                        for dr, wr in ((0, 1.0 - fr), (1, fr)):
                            w = wbg * wr
                            idx = base + (db * D2 + dg * DIM + dr)
                            acc0 += w * plsc.load_gather(lut_v, [idx])
                            acc1 += w * plsc.load_gather(lut_v, [idx + D3])
                            acc2 += w * plsc.load_gather(
                                lut_v, [idx + 2 * D3])
                out_v[pl.ds(s, L)] = acc0
                out_v[pl.ds(CHUNK + s, L)] = acc1
                out_v[pl.ds(2 * CHUNK + s, L)] = acc2

            def vec_body(vi, _):
                do_vreg(vi * L)
                return 0

            lax.fori_loop(0, CHUNK // L, vec_body, 0)

        issue_in(0, in0, is0)

        def body2(ci2, _):
            for b in range(2):
                ci = ci2 * 2 + b

                @pl.when(ci + 1 < n_chunks)
                def _():
                    issue_in(ci + 1, ins[1 - b], in_sems[1 - b])

                wait_in(ins[b], in_sems[b])

                @pl.when(ci2 >= 1)
                def _():
                    wait_out(obs[b], out_sems[b])

                compute(ins[b], obs[b])
                issue_out(ci, obs[b], out_sems[b])
            return 0

        lax.fori_loop(0, n_chunks // 2, body2, 0)
        wait_out(ob0, os0)
        wait_out(ob1, os1)

    return sc_kernel


def kernel(x, LUT):
    B, C, H, W = x.shape
    n_px = H * W
    xr = x.reshape(B * C * n_px)
    lut_flat = LUT.reshape(C * LUT.shape[1] ** 3)
    lut_pad = jnp.pad(lut_flat, (0, LUT_PAD - lut_flat.shape[0]))
    out = _make_sc_kernel(B * C, n_px)(xr, lut_pad)
    return out.reshape(B, C, H, W)
